# baseline (speedup 1.0000x reference)
import functools

import jax
import jax.numpy as jnp
from jax import lax
from jax.experimental import pallas as pl
from jax.experimental.pallas import tpu as pltpu
from jax.experimental.pallas import tpu_sc as plsc

N = 10000
E = 320000
D_IN = 128
D_H1 = 128
D_H2 = 64

NC = 2
NS = 16
NW = NC * NS

CHUNK = 128
K0 = 84
K1 = 76
IDX_ROWS = NS * (K0 + K1) + (K0 - K1)
E_PAD = IDX_ROWS * CHUNK

N_PAD = 10240
ROWS_PER_TILE = N_PAD // NS

_MESH = plsc.VectorSubcoreMesh(
    core_axis_name="c", subcore_axis_name="s", num_cores=NC, num_subcores=NS
)


def _chunk_split(c, s):
    nch = jnp.where(c == 0, K0, K1)
    base = jnp.where(c == 0, s * K0, NS * K0 + s * K1)
    return nch, base


def _deg_body(dst_hbm, ones_hbm, zeros_hbm, out_hbm, idx_v, ones_v, sem,
              deg_sp):
    c = lax.axis_index("c")
    s = lax.axis_index("s")
    nch, base = _chunk_split(c, s)
    row0 = s * ROWS_PER_TILE
    pltpu.sync_copy(zeros_hbm, deg_sp.at[pl.ds(row0, ROWS_PER_TILE)])
    pltpu.sync_copy(ones_hbm, ones_v)
    pltpu.sync_copy(dst_hbm.at[pl.ds(base, K0)], idx_v)
    plsc.subcore_barrier()

    GRP = 4

    def group(g, carry):
        j0 = g * GRP
        for b in range(GRP):
            pltpu.async_copy(ones_v, deg_sp.at[idx_v.at[j0 + b]], sem,
                             add=True)
        for b in range(GRP):
            pltpu.make_async_copy(ones_v, deg_sp.at[idx_v.at[j0 + b]],
                                  sem).wait()
        return carry

    lax.fori_loop(0, nch // GRP, group, 0)
    plsc.subcore_barrier()
    pltpu.sync_copy(
        deg_sp.at[pl.ds(row0, ROWS_PER_TILE)],
        out_hbm.at[pl.ds(c * N_PAD + row0, ROWS_PER_TILE)],
    )


_SC_PARAMS = pltpu.CompilerParams(use_tc_tiling_on_sc=False)

_sc_deg = pl.kernel(
    _deg_body,
    out_type=jax.ShapeDtypeStruct((NC * N_PAD, 16), jnp.float32),
    mesh=_MESH,
    compiler_params=_SC_PARAMS,
    scratch_types=[
        pltpu.VMEM((K0, CHUNK), jnp.int32),
        pltpu.VMEM((CHUNK, 16), jnp.float32),
        pltpu.SemaphoreType.DMA,
        pltpu.VMEM_SHARED((N_PAD, 16), jnp.float32),
    ],
)


def _agg_body(g_hbm, src_hbm, dst_hbm, zeros_hbm, out_hbm,
              sidx, didx0, didx1, rows0, rows1, gsem0, gsem1, dsem0, dsem1,
              acc_sp):
    c = lax.axis_index("c")
    s = lax.axis_index("s")
    nch, base = _chunk_split(c, s)
    row0 = s * ROWS_PER_TILE
    pltpu.sync_copy(zeros_hbm, acc_sp.at[pl.ds(row0, ROWS_PER_TILE)])
    pltpu.sync_copy(src_hbm.at[pl.ds(base, K0)], sidx)
    plsc.subcore_barrier()

    rows = (rows0, rows1)
    didx = (didx0, didx1)
    gsem = (gsem0, gsem1)
    dsem = (dsem0, dsem1)
    pltpu.async_copy(g_hbm.at[sidx.at[0]], rows[0], gsem[0])
    pltpu.async_copy(dst_hbm.at[0 + base], didx[0], dsem[0])

    def step2(g, carry):
        for b in range(2):
            j = g * 2 + b
            nb = 1 - b

            @pl.when(j + 1 < nch)
            def _():
                pltpu.async_copy(g_hbm.at[sidx.at[j + 1]], rows[nb], gsem[nb])
                pltpu.async_copy(dst_hbm.at[j + 1 + base], didx[nb], dsem[nb])

            pltpu.make_async_copy(g_hbm.at[sidx.at[j]], rows[b],
                                  gsem[b]).wait()
            pltpu.make_async_copy(dst_hbm.at[j + base], didx[b],
                                  dsem[b]).wait()
            pltpu.sync_copy(rows[b], acc_sp.at[didx[b]], add=True)
        return carry

    lax.fori_loop(0, nch // 2, step2, 0)
    plsc.subcore_barrier()
    pltpu.sync_copy(
        acc_sp.at[pl.ds(row0, ROWS_PER_TILE)],
        out_hbm.at[pl.ds(c * N_PAD + row0, ROWS_PER_TILE)],
    )


def _make_agg(d):
    return pl.kernel(
        _agg_body,
        out_type=jax.ShapeDtypeStruct((NC * N_PAD, d), jnp.float32),
        mesh=_MESH,
        compiler_params=_SC_PARAMS,
        scratch_types=[
            pltpu.VMEM((K0, CHUNK), jnp.int32),
            pltpu.VMEM((CHUNK,), jnp.int32),
            pltpu.VMEM((CHUNK,), jnp.int32),
            pltpu.VMEM((CHUNK, d), jnp.float32),
            pltpu.VMEM((CHUNK, d), jnp.float32),
            pltpu.SemaphoreType.DMA,
            pltpu.SemaphoreType.DMA,
            pltpu.SemaphoreType.DMA,
            pltpu.SemaphoreType.DMA,
            pltpu.VMEM_SHARED((N_PAD, d), jnp.float32),
        ],
    )


_sc_agg1 = _make_agg(D_H1)
_sc_agg2 = _make_agg(D_H2)


def _dis_from_parts(deg_ref):
    dp = deg_ref[...]
    deg = dp[:N_PAD, 0:1] + dp[N_PAD:, 0:1] + 1.0
    rows = lax.broadcasted_iota(jnp.int32, (N_PAD, 1), 0)
    return jnp.where(rows < N, lax.rsqrt(deg), 0.0), rows


def _tmm_body(x_ref, w1_ref, h1_ref):
    h1_ref[...] = jnp.dot(x_ref[...], w1_ref[...],
                          preferred_element_type=jnp.float32)


def _tca_body(deg_ref, h1_ref, g1_ref):
    dis, _ = _dis_from_parts(deg_ref)
    g1_ref[...] = h1_ref[...] * dis


def _bn_relu(out, rows, gamma, beta):
    out = jnp.where(rows < N, out, 0.0)
    mean = jnp.sum(out, axis=0, keepdims=True) * (1.0 / N)
    cent = jnp.where(rows < N, out - mean, 0.0)
    var = jnp.sum(cent * cent, axis=0, keepdims=True) * (1.0 / N)
    h = cent * lax.rsqrt(var + 1e-5) * gamma + beta
    return jnp.where(rows < N, jnp.maximum(h, 0.0), 0.0)


def _tcb_body(deg_ref, acc_ref, g1_ref, b1_ref, gm1_ref, bt1_ref, w2_ref,
              g2_ref):
    dis, rows = _dis_from_parts(deg_ref)
    acc = acc_ref[:N_PAD, :] + acc_ref[N_PAD:, :] + g1_ref[...]
    out = acc * dis + b1_ref[...]
    h = _bn_relu(out, rows, gm1_ref[...], bt1_ref[...])
    h2 = jnp.dot(h, w2_ref[...], preferred_element_type=jnp.float32)
    g2_ref[...] = h2 * dis


def _tcc_body(deg_ref, acc_ref, g2_ref, b2_ref, gm2_ref, bt2_ref, out_ref):
    dis, rows = _dis_from_parts(deg_ref)
    acc = acc_ref[:N_PAD, :] + acc_ref[N_PAD:, :] + g2_ref[...]
    out = acc * dis + b2_ref[...]
    out_ref[...] = _bn_relu(out, rows, gm2_ref[...], bt2_ref[...])[:N]


_tc_mm = pl.pallas_call(
    _tmm_body,
    out_shape=jax.ShapeDtypeStruct((N_PAD, D_H1), jnp.float32),
)
_tc_a = pl.pallas_call(
    _tca_body,
    out_shape=jax.ShapeDtypeStruct((N_PAD, D_H1), jnp.float32),
)
_tc_b = pl.pallas_call(
    _tcb_body,
    out_shape=jax.ShapeDtypeStruct((N_PAD, D_H2), jnp.float32),
)
_tc_c = pl.pallas_call(
    _tcc_body,
    out_shape=jax.ShapeDtypeStruct((N, D_H2), jnp.float32),
)


@jax.jit
def kernel(x, edge_index, W1, b1, gamma1, beta1, W2, b2, gamma2, beta2):
    src = edge_index[0].astype(jnp.int32)
    dst = edge_index[1].astype(jnp.int32)
    pad_src = (N + jnp.arange(E_PAD - E, dtype=jnp.int32) % (N_PAD - N))
    pad_dst = jnp.full((E_PAD - E,), N, dtype=jnp.int32)
    src_pad = jnp.concatenate([src, pad_src]).reshape(IDX_ROWS, CHUNK)
    dst_pad = jnp.concatenate([dst, pad_dst]).reshape(IDX_ROWS, CHUNK)
    x_pad = jnp.zeros((N_PAD, D_IN), jnp.float32).at[:N].set(x)

    ones16 = jnp.ones((CHUNK, 16), jnp.float32)
    zeros16 = jnp.zeros((ROWS_PER_TILE, 16), jnp.float32)
    zeros1 = jnp.zeros((ROWS_PER_TILE, D_H1), jnp.float32)
    zeros2 = jnp.zeros((ROWS_PER_TILE, D_H2), jnp.float32)

    h1 = _tc_mm(x_pad, W1)
    deg_parts = _sc_deg(dst_pad, ones16, zeros16)
    g1 = _tc_a(deg_parts, h1)
    acc1 = _sc_agg1(g1, src_pad, dst_pad, zeros1)
    g2 = _tc_b(deg_parts, acc1, g1, b1.reshape(1, D_H1),
               gamma1.reshape(1, D_H1), beta1.reshape(1, D_H1), W2)
    acc2 = _sc_agg2(g2, src_pad, dst_pad, zeros2)
    return _tc_c(deg_parts, acc2, g2, b2.reshape(1, D_H2),
                 gamma2.reshape(1, D_H2), beta2.reshape(1, D_H2))


# speedup vs baseline: 1.0091x; 1.0091x over previous
import functools

import jax
import jax.numpy as jnp
from jax import lax
from jax.experimental import pallas as pl
from jax.experimental.pallas import tpu as pltpu
from jax.experimental.pallas import tpu_sc as plsc

N = 10000
E = 320000
D_IN = 128
D_H1 = 128
D_H2 = 64

NC = 2
NS = 16
NW = NC * NS

CHUNK = 128
K0 = 84
K1 = 76
IDX_ROWS = NS * (K0 + K1) + (K0 - K1)
E_PAD = IDX_ROWS * CHUNK

N_PAD = 10240
ROWS_PER_TILE = N_PAD // NS

_MESH = plsc.VectorSubcoreMesh(
    core_axis_name="c", subcore_axis_name="s", num_cores=NC, num_subcores=NS
)


def _chunk_split(c, s):
    nch = jnp.where(c == 0, K0, K1)
    base = jnp.where(c == 0, s * K0, NS * K0 + s * K1)
    return nch, base


def _deg_body(dst_hbm, ones_hbm, zeros_hbm, out_hbm, idx_v, ones_v, sem,
              deg_sp):
    c = lax.axis_index("c")
    s = lax.axis_index("s")
    nch, base = _chunk_split(c, s)
    row0 = s * ROWS_PER_TILE
    pltpu.sync_copy(zeros_hbm, deg_sp.at[pl.ds(row0, ROWS_PER_TILE)])
    pltpu.sync_copy(ones_hbm, ones_v)
    pltpu.sync_copy(dst_hbm.at[pl.ds(base, K0)], idx_v)
    plsc.subcore_barrier()

    GRP = 4

    def group(g, carry):
        j0 = g * GRP
        for b in range(GRP):
            pltpu.async_copy(ones_v, deg_sp.at[idx_v.at[j0 + b]], sem,
                             add=True)
        for b in range(GRP):
            pltpu.make_async_copy(ones_v, deg_sp.at[idx_v.at[j0 + b]],
                                  sem).wait()
        return carry

    lax.fori_loop(0, nch // GRP, group, 0)
    plsc.subcore_barrier()
    pltpu.sync_copy(
        deg_sp.at[pl.ds(row0, ROWS_PER_TILE)],
        out_hbm.at[pl.ds(c * N_PAD + row0, ROWS_PER_TILE)],
    )


_SC_PARAMS = pltpu.CompilerParams(use_tc_tiling_on_sc=False)

_sc_deg = pl.kernel(
    _deg_body,
    out_type=jax.ShapeDtypeStruct((NC * N_PAD, 16), jnp.float32),
    mesh=_MESH,
    compiler_params=_SC_PARAMS,
    scratch_types=[
        pltpu.VMEM((K0, CHUNK), jnp.int32),
        pltpu.VMEM((CHUNK, 16), jnp.float32),
        pltpu.SemaphoreType.DMA,
        pltpu.VMEM_SHARED((N_PAD, 16), jnp.float32),
    ],
)


def _agg_body(g_hbm, src_hbm, dst_hbm, zeros_hbm, out_hbm,
              sidx, didx0, didx1, rows0, rows1, gsem0, gsem1, dsem0, dsem1,
              acc_sp):
    c = lax.axis_index("c")
    s = lax.axis_index("s")
    nch, base = _chunk_split(c, s)
    row0 = s * ROWS_PER_TILE
    pltpu.sync_copy(zeros_hbm, acc_sp.at[pl.ds(row0, ROWS_PER_TILE)])
    pltpu.sync_copy(src_hbm.at[pl.ds(base, K0)], sidx)
    plsc.subcore_barrier()

    rows = (rows0, rows1)
    didx = (didx0, didx1)
    gsem = (gsem0, gsem1)
    dsem = (dsem0, dsem1)
    pltpu.async_copy(g_hbm.at[sidx.at[0]], rows[0], gsem[0])
    pltpu.async_copy(dst_hbm.at[0 + base], didx[0], dsem[0])

    def step2(g, carry):
        for b in range(2):
            j = g * 2 + b
            nb = 1 - b

            @pl.when(j + 1 < nch)
            def _():
                pltpu.async_copy(g_hbm.at[sidx.at[j + 1]], rows[nb], gsem[nb])
                pltpu.async_copy(dst_hbm.at[j + 1 + base], didx[nb], dsem[nb])

            pltpu.make_async_copy(g_hbm.at[sidx.at[j]], rows[b],
                                  gsem[b]).wait()
            pltpu.make_async_copy(dst_hbm.at[j + base], didx[b],
                                  dsem[b]).wait()
            pltpu.sync_copy(rows[b], acc_sp.at[didx[b]], add=True)
        return carry

    lax.fori_loop(0, nch // 2, step2, 0)
    plsc.subcore_barrier()
    pltpu.sync_copy(
        acc_sp.at[pl.ds(row0, ROWS_PER_TILE)],
        out_hbm.at[pl.ds(c * N_PAD + row0, ROWS_PER_TILE)],
    )


def _make_agg(d):
    return pl.kernel(
        _agg_body,
        out_type=jax.ShapeDtypeStruct((NC * N_PAD, d), jnp.float32),
        mesh=_MESH,
        compiler_params=_SC_PARAMS,
        scratch_types=[
            pltpu.VMEM((K0, CHUNK), jnp.int32),
            pltpu.VMEM((CHUNK,), jnp.int32),
            pltpu.VMEM((CHUNK,), jnp.int32),
            pltpu.VMEM((CHUNK, d), jnp.float32),
            pltpu.VMEM((CHUNK, d), jnp.float32),
            pltpu.SemaphoreType.DMA,
            pltpu.SemaphoreType.DMA,
            pltpu.SemaphoreType.DMA,
            pltpu.SemaphoreType.DMA,
            pltpu.VMEM_SHARED((N_PAD, d), jnp.float32),
        ],
    )


_sc_agg1 = _make_agg(D_H1)
_sc_agg2 = _make_agg(D_H2)


def _dis_from_parts(deg_ref):
    dp = deg_ref[...]
    deg = dp[:N_PAD, 0:1] + dp[N_PAD:, 0:1] + 1.0
    rows = lax.broadcasted_iota(jnp.int32, (N_PAD, 1), 0)
    return jnp.where(rows < N, lax.rsqrt(deg), 0.0), rows


def _tca_body(deg_ref, x_ref, w1_ref, g1_ref):
    dis, _ = _dis_from_parts(deg_ref)
    h = jnp.dot(x_ref[...], w1_ref[...], preferred_element_type=jnp.float32)
    g1_ref[...] = h * dis


def _bn_relu(out, rows, gamma, beta):
    out = jnp.where(rows < N, out, 0.0)
    mean = jnp.sum(out, axis=0, keepdims=True) * (1.0 / N)
    cent = jnp.where(rows < N, out - mean, 0.0)
    var = jnp.sum(cent * cent, axis=0, keepdims=True) * (1.0 / N)
    h = cent * lax.rsqrt(var + 1e-5) * gamma + beta
    return jnp.where(rows < N, jnp.maximum(h, 0.0), 0.0)


def _tcb_body(deg_ref, acc_ref, g1_ref, b1_ref, gm1_ref, bt1_ref, w2_ref,
              g2_ref):
    dis, rows = _dis_from_parts(deg_ref)
    acc = acc_ref[:N_PAD, :] + acc_ref[N_PAD:, :] + g1_ref[...]
    out = acc * dis + b1_ref[...]
    h = _bn_relu(out, rows, gm1_ref[...], bt1_ref[...])
    h2 = jnp.dot(h, w2_ref[...], preferred_element_type=jnp.float32)
    g2_ref[...] = h2 * dis


def _tcc_body(deg_ref, acc_ref, g2_ref, b2_ref, gm2_ref, bt2_ref, out_ref):
    dis, rows = _dis_from_parts(deg_ref)
    acc = acc_ref[:N_PAD, :] + acc_ref[N_PAD:, :] + g2_ref[...]
    out = acc * dis + b2_ref[...]
    out_ref[...] = _bn_relu(out, rows, gm2_ref[...], bt2_ref[...])[:N]


_tc_a = pl.pallas_call(
    _tca_body,
    out_shape=jax.ShapeDtypeStruct((N_PAD, D_H1), jnp.float32),
)
_tc_b = pl.pallas_call(
    _tcb_body,
    out_shape=jax.ShapeDtypeStruct((N_PAD, D_H2), jnp.float32),
)
_tc_c = pl.pallas_call(
    _tcc_body,
    out_shape=jax.ShapeDtypeStruct((N, D_H2), jnp.float32),
)


@jax.jit
def kernel(x, edge_index, W1, b1, gamma1, beta1, W2, b2, gamma2, beta2):
    src = edge_index[0].astype(jnp.int32)
    dst = edge_index[1].astype(jnp.int32)
    pad_src = (N + jnp.arange(E_PAD - E, dtype=jnp.int32) % (N_PAD - N))
    pad_dst = jnp.full((E_PAD - E,), N, dtype=jnp.int32)
    src_pad = jnp.concatenate([src, pad_src]).reshape(IDX_ROWS, CHUNK)
    dst_pad = jnp.concatenate([dst, pad_dst]).reshape(IDX_ROWS, CHUNK)
    x_pad = jnp.zeros((N_PAD, D_IN), jnp.float32).at[:N].set(x)

    ones16 = jnp.ones((CHUNK, 16), jnp.float32)
    zeros16 = jnp.zeros((ROWS_PER_TILE, 16), jnp.float32)
    zeros1 = jnp.zeros((ROWS_PER_TILE, D_H1), jnp.float32)
    zeros2 = jnp.zeros((ROWS_PER_TILE, D_H2), jnp.float32)

    deg_parts = _sc_deg(dst_pad, ones16, zeros16)
    g1 = _tc_a(deg_parts, x_pad, W1)
    acc1 = _sc_agg1(g1, src_pad, dst_pad, zeros1)
    g2 = _tc_b(deg_parts, acc1, g1, b1.reshape(1, D_H1),
               gamma1.reshape(1, D_H1), beta1.reshape(1, D_H1), W2)
    acc2 = _sc_agg2(g2, src_pad, dst_pad, zeros2)
    return _tc_c(deg_parts, acc2, g2, b2.reshape(1, D_H2),
                 gamma2.reshape(1, D_H2), beta2.reshape(1, D_H2))
